# baseline (speedup 1.0000x reference)
import functools

import jax
import jax.numpy as jnp
from jax import lax
from jax.experimental import pallas as pl
from jax.experimental.pallas import tpu as pltpu
from jax.experimental.pallas import tpu_sc as plsc

_NC = 2
_NS = 16
_NW = _NC * _NS
_LANES = 16
_CH = 64


def _first_two(Ht):
    nz = Ht > 0
    cum = jnp.cumsum(nz.astype(jnp.int32), axis=1)
    sel0 = nz & (cum == 1)
    sel1 = nz & (cum == 2)
    a0 = jnp.argmax(sel0, axis=1).astype(jnp.int32)
    a1 = jnp.argmax(sel1, axis=1).astype(jnp.int32)
    w0 = jnp.sum(jnp.where(sel0, Ht, 0.0), axis=1).astype(jnp.float32)
    w1 = jnp.sum(jnp.where(sel1, Ht, 0.0), axis=1).astype(jnp.float32)
    return a0, a1, w0, w1


def _lane_tables(H_z, H_x, qubit_rows, qubit_cols, qubit_src_idx,
                 z_stab_rows, z_stab_cols, z_stab_src_idx,
                 x_stab_rows, x_stab_cols, x_stab_src_idx, new):
    n_z = H_z.shape[0]
    cells = new * new

    za0, za1, zw0, zw1 = _first_two(H_z.T)
    xa0, xa1, xw0, xw1 = _first_two(H_x.T)
    q = qubit_src_idx

    i32 = jnp.int32
    f32 = jnp.float32
    n_st = z_stab_src_idx.shape[0] + x_stab_src_idx.shape[0]

    i0 = jnp.concatenate([
        z_stab_src_idx, x_stab_src_idx + n_z,
        za0[q], xa0[q] + n_z,
    ]).astype(i32)
    i1 = jnp.concatenate([
        jnp.zeros((n_st,), i32),
        za1[q], xa1[q] + n_z,
    ]).astype(i32)
    w0 = jnp.concatenate([
        jnp.full((n_st,), 2.0, f32),
        0.25 * zw0[q], 0.25 * xw0[q],
    ]).astype(f32)
    w1 = jnp.concatenate([
        jnp.zeros((n_st,), f32),
        0.25 * zw1[q], 0.25 * xw1[q],
    ]).astype(f32)
    bias = jnp.concatenate([
        jnp.full((n_st,), -1.0, f32),
        jnp.zeros((2 * q.shape[0],), f32),
    ]).astype(f32)
    col = jnp.concatenate([
        2 * cells + z_stab_rows * new + z_stab_cols,
        3 * cells + x_stab_rows * new + x_stab_cols,
        4 * cells + qubit_rows * new + qubit_cols,
        5 * cells + qubit_rows * new + qubit_cols,
    ]).astype(i32)

    total = i0.shape[0]
    ngrp = -(-total // _LANES)
    pad = ngrp * _LANES - total

    def padded(a, val):
        return jnp.pad(a, (0, pad), constant_values=val)

    msk = padded(jnp.ones((total,), i32), 0)
    return (padded(i0, 0), padded(i1, 0), padded(w0, 0.0), padded(w1, 0.0),
            padded(bias, 0.0), padded(col, 0), msk, ngrp)


def _make_sc_call(B, SW, out_cols, ngrp, glanes):
    rows_w = B // _NW
    nchunk = rows_w // _CH
    mesh = plsc.VectorSubcoreMesh(core_axis_name="c", subcore_axis_name="s")

    @functools.partial(
        pl.kernel,
        out_type=jax.ShapeDtypeStruct((B, out_cols), jnp.float32),
        mesh=mesh,
        compiler_params=pltpu.CompilerParams(
            use_tc_tiling_on_sc=False, needs_layout_passes=False),
        scratch_types=[
            pltpu.VMEM((rows_w, SW), jnp.float32),
            pltpu.VMEM((_CH, out_cols), jnp.float32),
            pltpu.VMEM((_CH, out_cols), jnp.float32),
            pltpu.VMEM((glanes,), jnp.int32),
            pltpu.VMEM((glanes,), jnp.int32),
            pltpu.VMEM((glanes,), jnp.float32),
            pltpu.VMEM((glanes,), jnp.float32),
            pltpu.VMEM((glanes,), jnp.float32),
            pltpu.VMEM((glanes,), jnp.int32),
            pltpu.VMEM((glanes,), jnp.int32),
            pltpu.SemaphoreType.DMA,
            pltpu.SemaphoreType.DMA,
            pltpu.SemaphoreType.DMA,
        ],
    )
    def sc_call(synd, i0_h, i1_h, w0_h, w1_h, b_h, col_h, m_h, out,
                synd_v, buf0, buf1, i0_v, i1_v, w0_v, w1_v, b_v, col_v, m_v,
                sem_in, sem_a, sem_b):
        wid = lax.axis_index("s") * _NC + lax.axis_index("c")
        base = wid * rows_w

        cp_in = pltpu.async_copy(synd.at[pl.ds(base, rows_w)], synd_v, sem_in)
        pltpu.sync_copy(i0_h, i0_v)
        pltpu.sync_copy(i1_h, i1_v)
        pltpu.sync_copy(w0_h, w0_v)
        pltpu.sync_copy(w1_h, w1_v)
        pltpu.sync_copy(b_h, b_v)
        pltpu.sync_copy(col_h, col_v)
        pltpu.sync_copy(m_h, m_v)

        zvec = jnp.zeros((_LANES,), jnp.float32)
        iota = lax.iota(jnp.int32, _LANES)
        nfull = out_cols // _LANES
        ntail = out_cols - nfull * _LANES
        tail_cols = nfull * _LANES + iota
        tail_msk = iota < ntail

        def zero_rows(buf):
            def body(r, carry):
                for i in range(nfull):
                    buf[r, pl.ds(i * _LANES, _LANES)] = zvec
                if ntail:
                    plsc.store_scatter(
                        buf, [jnp.full((_LANES,), r, jnp.int32), tail_cols],
                        zvec, mask=tail_msk)
                return carry
            lax.fori_loop(0, _CH, body, 0)

        zero_rows(buf0)
        zero_rows(buf1)

        gi0 = [i0_v[pl.ds(g * _LANES, _LANES)] for g in range(ngrp)]
        gi1 = [i1_v[pl.ds(g * _LANES, _LANES)] for g in range(ngrp)]
        gw0 = [w0_v[pl.ds(g * _LANES, _LANES)] for g in range(ngrp)]
        gw1 = [w1_v[pl.ds(g * _LANES, _LANES)] for g in range(ngrp)]
        gb = [b_v[pl.ds(g * _LANES, _LANES)] for g in range(ngrp)]
        gcol = [col_v[pl.ds(g * _LANES, _LANES)] for g in range(ngrp)]
        gmsk = [m_v[pl.ds(g * _LANES, _LANES)] != 0 for g in range(ngrp)]

        cp_in.wait()

        def do_chunk(c, buf):
            def body(r, carry):
                lr = c * _CH + r
                rsplat = jnp.full((_LANES,), r, jnp.int32)
                lrsplat = jnp.full((_LANES,), lr, jnp.int32)
                for g in range(ngrp):
                    v0 = plsc.load_gather(synd_v, [lrsplat, gi0[g]])
                    v1 = plsc.load_gather(synd_v, [lrsplat, gi1[g]])
                    val = v0 * gw0[g] + v1 * gw1[g] + gb[g]
                    plsc.store_scatter(buf, [rsplat, gcol[g]], val,
                                       mask=gmsk[g])
                return carry
            lax.fori_loop(0, _CH, body, 0)

        pending = [None, None]
        for c in range(nchunk):
            p = c & 1
            buf = buf0 if p == 0 else buf1
            sem = sem_a if p == 0 else sem_b
            if pending[p] is not None:
                pending[p].wait()
            do_chunk(c, buf)
            pending[p] = pltpu.async_copy(
                buf, out.at[pl.ds(base + c * _CH, _CH)], sem)
        for p in (0, 1):
            if pending[p] is not None:
                pending[p].wait()

    return sc_call


def kernel(syndrome, H_z, H_x, qubit_rows, qubit_cols, qubit_src_idx,
           z_stab_rows, z_stab_cols, z_stab_src_idx,
           x_stab_rows, x_stab_cols, x_stab_src_idx):
    B, SW = syndrome.shape
    NQ = H_z.shape[1]
    L = int(round(NQ ** 0.5))
    new = 2 * L - 1
    out_cols = 6 * new * new

    i0, i1, w0, w1, bias, col, msk, ngrp = _lane_tables(
        H_z, H_x, qubit_rows, qubit_cols, qubit_src_idx,
        z_stab_rows, z_stab_cols, z_stab_src_idx,
        x_stab_rows, x_stab_cols, x_stab_src_idx, new)
    glanes = ngrp * _LANES

    sc_call = _make_sc_call(B, SW, out_cols, ngrp, glanes)
    flat = sc_call(syndrome.astype(jnp.float32), i0, i1, w0, w1, bias, col,
                   msk)
    return flat.reshape(B, 6, new, new).astype(syndrome.dtype)
